# baseline (speedup 1.0000x reference)
import functools

import jax
import jax.numpy as jnp
from jax import lax
from jax.experimental import pallas as pl
from jax.experimental.pallas import tpu as pltpu
from jax.experimental.pallas import tpu_sc as plsc

D = 256
K = 8192
N = 8192
NB = 8
KB = 8
BN = N // NB
BK = K // KB

_SC_NC = 2
_SC_NS = 16
_SC_NW = _SC_NC * _SC_NS
_B_PER_W = N // _SC_NW


def _argmin_body(x_ref, w2_ref, ids_ref, idx_ref, loss_ref,
                 mv_s, mi_s, s1_s, acc_s):
    nb = pl.program_id(0)
    kb = pl.program_id(1)
    x = x_ref[...]
    w2 = w2_ref[...]
    s = lax.dot_general(w2, x, (((1,), (1,)), ((), ())),
                        preferred_element_type=jnp.float32)

    @pl.when(kb == 0)
    def _():
        s1 = jnp.sum(x * x, axis=1, keepdims=True)
        s1_s[...] = lax.transpose(s1, (1, 0))

    s2 = 0.25 * jnp.sum(w2 * w2, axis=1, keepdims=True)
    d = (s1_s[...] + s) + s2
    bmin = jnp.min(d, axis=0, keepdims=True)
    idsf = ids_ref[...]
    bidx = (jnp.min(jnp.where(d == bmin, idsf, jnp.float32(3e10)),
                    axis=0, keepdims=True)
            + (kb * BK).astype(jnp.float32))

    @pl.when(kb == 0)
    def _():
        mv_s[...] = bmin
        mi_s[...] = bidx

    @pl.when(kb > 0)
    def _():
        better = bmin < mv_s[...]
        mi_s[...] = jnp.where(better, bidx, mi_s[...])
        mv_s[...] = jnp.where(better, bmin, mv_s[...])

    @pl.when(kb == KB - 1)
    def _():
        idx_ref[...] = mi_s[...].astype(jnp.int32).reshape(1, 1, BN)
        rowsum = jnp.sum(mv_s[...])

        @pl.when(nb == 0)
        def _():
            acc_s[0, 0] = rowsum

        @pl.when(nb > 0)
        def _():
            acc_s[0, 0] = acc_s[0, 0] + rowsum

        @pl.when(nb == NB - 1)
        def _():
            loss = 0.25 * acc_s[0, 0] * (1.0 / (N * D))
            loss_ref[...] = jnp.full((1, 128), loss, jnp.float32)


def _onehot_body(idx_ref, enc_ref, perp_ref, cnt_s, ent_s):
    kb = pl.program_id(0)
    nb = pl.program_id(1)
    idx_row = idx_ref[...].reshape(1, BN)
    idx_col = lax.transpose(idx_row, (1, 0))
    ids = lax.broadcasted_iota(jnp.int32, (BN, BK), 1) + kb * BK
    onehot = (ids == idx_col).astype(jnp.float32)
    enc_ref[...] = onehot
    ones = jnp.ones((1, BN), jnp.float32)
    cnt = lax.dot_general(ones, onehot, (((1,), (0,)), ((), ())),
                          preferred_element_type=jnp.float32)

    @pl.when(nb == 0)
    def _():
        cnt_s[...] = cnt

    @pl.when(nb > 0)
    def _():
        cnt_s[...] = cnt_s[...] + cnt

    @pl.when(nb == NB - 1)
    def _():
        p = cnt_s[...] * (1.0 / N)
        ev = jnp.sum(p * jnp.log(p + 1e-10))

        @pl.when(kb == 0)
        def _():
            ent_s[0, 0] = ev

        @pl.when(kb > 0)
        def _():
            ent_s[0, 0] = ent_s[0, 0] + ev

        @pl.when(kb == KB - 1)
        def _():
            perp_ref[...] = jnp.exp(jnp.full((1, 128), -ent_s[0, 0],
                                             jnp.float32))


def _sc_gather(table_hbm, idx_hbm, out_hbm, idx_v, rows_v, sem):
    wid = lax.axis_index("s") * _SC_NC + lax.axis_index("c")
    base = wid * _B_PER_W
    pltpu.sync_copy(idx_hbm.at[pl.ds(base, _B_PER_W)], idx_v)
    pltpu.async_copy(table_hbm.at[idx_v], rows_v, sem).wait()
    pltpu.sync_copy(rows_v, out_hbm.at[pl.ds(base, _B_PER_W)])


def kernel(inputTensor, emb_weights, ema_w, ema_cluster_size):
    del ema_w, ema_cluster_size
    flat = inputTensor.reshape(-1, D)
    w2 = -2.0 * emb_weights
    ids_col = jnp.arange(BK, dtype=jnp.float32).reshape(BK, 1)

    idx3, loss_out = pl.pallas_call(
        _argmin_body,
        grid=(NB, KB),
        in_specs=[
            pl.BlockSpec((BN, D), lambda nb, kb: (nb, 0)),
            pl.BlockSpec((BK, D), lambda nb, kb: (kb, 0)),
            pl.BlockSpec((BK, 1), lambda nb, kb: (0, 0)),
        ],
        out_specs=[
            pl.BlockSpec((1, 1, BN), lambda nb, kb: (nb, 0, 0)),
            pl.BlockSpec((1, 128), lambda nb, kb: (0, 0)),
        ],
        out_shape=[
            jax.ShapeDtypeStruct((NB, 1, BN), jnp.int32),
            jax.ShapeDtypeStruct((1, 128), jnp.float32),
        ],
        scratch_shapes=[
            pltpu.VMEM((1, BN), jnp.float32),
            pltpu.VMEM((1, BN), jnp.float32),
            pltpu.VMEM((1, BN), jnp.float32),
            pltpu.SMEM((1, 1), jnp.float32),
        ],
    )(flat, w2, ids_col)

    idx_flat = idx3.reshape(N)

    sc_gather = functools.partial(
        pl.kernel,
        mesh=plsc.VectorSubcoreMesh(core_axis_name="c", subcore_axis_name="s"),
        out_type=jax.ShapeDtypeStruct((N, D), jnp.float32),
        scratch_types=[
            pltpu.VMEM((_B_PER_W,), jnp.int32),
            pltpu.VMEM((_B_PER_W, D), jnp.float32),
            pltpu.SemaphoreType.DMA,
        ],
    )(_sc_gather)
    quantized = sc_gather(emb_weights, idx_flat)

    enc, perp_out = pl.pallas_call(
        _onehot_body,
        grid=(KB, NB),
        in_specs=[
            pl.BlockSpec((1, 1, BN), lambda kb, nb: (nb, 0, 0)),
        ],
        out_specs=[
            pl.BlockSpec((BN, BK), lambda kb, nb: (nb, kb)),
            pl.BlockSpec((1, 128), lambda kb, nb: (0, 0)),
        ],
        out_shape=[
            jax.ShapeDtypeStruct((N, K), jnp.float32),
            jax.ShapeDtypeStruct((1, 128), jnp.float32),
        ],
        scratch_shapes=[
            pltpu.VMEM((1, BK), jnp.float32),
            pltpu.SMEM((1, 1), jnp.float32),
        ],
    )(idx3)

    loss = loss_out[0, 0]
    perplexity = perp_out[0, 0]
    quantized_st = quantized.reshape(inputTensor.shape)
    return (loss, quantized_st, perplexity, enc)


# speedup vs baseline: 1.0342x; 1.0342x over previous
import functools

import jax
import jax.numpy as jnp
from jax import lax
from jax.experimental import pallas as pl
from jax.experimental.pallas import tpu as pltpu
from jax.experimental.pallas import tpu_sc as plsc

D = 256
K = 8192
N = 8192
NB = 8
KB = 8
BN = N // NB
BK = K // KB

_SC_NC = 2
_SC_NS = 16
_SC_NW = _SC_NC * _SC_NS
_B_PER_W = N // _SC_NW


def _argmin_body(xt_ref, w2_ref, ids_ref, idx_ref, loss_ref,
                 mv_s, mi_s, s1_s, acc_s):
    nb = pl.program_id(0)
    kb = pl.program_id(1)
    xt = xt_ref[...]
    w2 = w2_ref[...]
    s = lax.dot_general(w2, xt, (((1,), (0,)), ((), ())),
                        preferred_element_type=jnp.float32)

    @pl.when(kb == 0)
    def _():
        s1_s[...] = jnp.sum(xt * xt, axis=0, keepdims=True)

    s2 = 0.25 * jnp.sum(w2 * w2, axis=1, keepdims=True)
    d = (s1_s[...] + s) + s2
    bmin = jnp.min(d, axis=0, keepdims=True)
    idsf = ids_ref[...]
    bidx = (jnp.min(jnp.where(d == bmin, idsf, jnp.float32(3e10)),
                    axis=0, keepdims=True)
            + (kb * BK).astype(jnp.float32))

    @pl.when(kb == 0)
    def _():
        mv_s[...] = bmin
        mi_s[...] = bidx

    @pl.when(kb > 0)
    def _():
        better = bmin < mv_s[...]
        mi_s[...] = jnp.where(better, bidx, mi_s[...])
        mv_s[...] = jnp.where(better, bmin, mv_s[...])

    @pl.when(kb == KB - 1)
    def _():
        idx_ref[...] = mi_s[...].astype(jnp.int32).reshape(1, 1, BN)
        rowsum = jnp.sum(mv_s[...])

        @pl.when(nb == 0)
        def _():
            acc_s[0, 0] = rowsum

        @pl.when(nb > 0)
        def _():
            acc_s[0, 0] = acc_s[0, 0] + rowsum

        @pl.when(nb == NB - 1)
        def _():
            loss = 0.25 * acc_s[0, 0] * (1.0 / (N * D))
            loss_ref[...] = jnp.full((1, 128), loss, jnp.float32)


def _onehot_body(idx_ref, enc_ref, perp_ref, cnt_s, ent_s):
    kb = pl.program_id(0)
    nb = pl.program_id(1)
    idx_row = idx_ref[...].reshape(1, BN)
    idx_col = lax.transpose(idx_row, (1, 0))
    ids = lax.broadcasted_iota(jnp.int32, (BN, BK), 1) + kb * BK
    onehot = (ids == idx_col).astype(jnp.float32)
    enc_ref[...] = onehot
    cnt = jnp.sum(onehot, axis=0, keepdims=True)

    @pl.when(nb == 0)
    def _():
        cnt_s[...] = cnt

    @pl.when(nb > 0)
    def _():
        cnt_s[...] = cnt_s[...] + cnt

    @pl.when(nb == NB - 1)
    def _():
        p = cnt_s[...] * (1.0 / N)
        ev = jnp.sum(p * jnp.log(p + 1e-10))

        @pl.when(kb == 0)
        def _():
            ent_s[0, 0] = ev

        @pl.when(kb > 0)
        def _():
            ent_s[0, 0] = ent_s[0, 0] + ev

        @pl.when(kb == KB - 1)
        def _():
            perp_ref[...] = jnp.exp(jnp.full((1, 128), -ent_s[0, 0],
                                             jnp.float32))


def _sc_gather(table_hbm, idx_hbm, out_hbm, idx_v, rows_v, sem):
    wid = lax.axis_index("s") * _SC_NC + lax.axis_index("c")
    base = wid * _B_PER_W
    pltpu.sync_copy(idx_hbm.at[pl.ds(base, _B_PER_W)], idx_v)
    pltpu.async_copy(table_hbm.at[idx_v], rows_v, sem).wait()
    pltpu.sync_copy(rows_v, out_hbm.at[pl.ds(base, _B_PER_W)])


def kernel(inputTensor, emb_weights, ema_w, ema_cluster_size):
    del ema_w, ema_cluster_size
    flat = inputTensor.reshape(-1, D)
    xt = flat.T
    w2 = -2.0 * emb_weights
    ids_full = jnp.broadcast_to(
        jnp.arange(BK, dtype=jnp.float32).reshape(BK, 1), (BK, BN))

    idx3, loss_out = pl.pallas_call(
        _argmin_body,
        grid=(NB, KB),
        in_specs=[
            pl.BlockSpec((D, BN), lambda nb, kb: (0, nb)),
            pl.BlockSpec((BK, D), lambda nb, kb: (kb, 0)),
            pl.BlockSpec((BK, BN), lambda nb, kb: (0, 0)),
        ],
        out_specs=[
            pl.BlockSpec((1, 1, BN), lambda nb, kb: (nb, 0, 0)),
            pl.BlockSpec((1, 128), lambda nb, kb: (0, 0)),
        ],
        out_shape=[
            jax.ShapeDtypeStruct((NB, 1, BN), jnp.int32),
            jax.ShapeDtypeStruct((1, 128), jnp.float32),
        ],
        scratch_shapes=[
            pltpu.VMEM((1, BN), jnp.float32),
            pltpu.VMEM((1, BN), jnp.float32),
            pltpu.VMEM((1, BN), jnp.float32),
            pltpu.SMEM((1, 1), jnp.float32),
        ],
    )(xt, w2, ids_full)

    idx_flat = idx3.reshape(N)

    sc_gather = functools.partial(
        pl.kernel,
        mesh=plsc.VectorSubcoreMesh(core_axis_name="c", subcore_axis_name="s"),
        out_type=jax.ShapeDtypeStruct((N, D), jnp.float32),
        scratch_types=[
            pltpu.VMEM((_B_PER_W,), jnp.int32),
            pltpu.VMEM((_B_PER_W, D), jnp.float32),
            pltpu.SemaphoreType.DMA,
        ],
    )(_sc_gather)
    quantized = sc_gather(emb_weights, idx_flat)

    enc, perp_out = pl.pallas_call(
        _onehot_body,
        grid=(KB, NB),
        in_specs=[
            pl.BlockSpec((1, 1, BN), lambda kb, nb: (nb, 0, 0)),
        ],
        out_specs=[
            pl.BlockSpec((BN, BK), lambda kb, nb: (nb, kb)),
            pl.BlockSpec((1, 128), lambda kb, nb: (0, 0)),
        ],
        out_shape=[
            jax.ShapeDtypeStruct((N, K), jnp.float32),
            jax.ShapeDtypeStruct((1, 128), jnp.float32),
        ],
        scratch_shapes=[
            pltpu.VMEM((1, BK), jnp.float32),
            pltpu.SMEM((1, 1), jnp.float32),
        ],
    )(idx3)

    loss = loss_out[0, 0]
    perplexity = perp_out[0, 0]
    quantized_st = quantized.reshape(inputTensor.shape)
    return (loss, quantized_st, perplexity, enc)


# speedup vs baseline: 1.1255x; 1.0883x over previous
import functools

import jax
import jax.numpy as jnp
from jax import lax
from jax.experimental import pallas as pl
from jax.experimental.pallas import tpu as pltpu
from jax.experimental.pallas import tpu_sc as plsc

D = 256
K = 8192
N = 8192
NB = 8
KB = 8
BN = N // NB
BK = K // KB

_SC_NC = 2
_SC_NS = 16
_SC_NW = _SC_NC * _SC_NS
_B_PER_W = N // _SC_NW


def _argmin_body(xt_ref, w2_ref, idx_ref, loss_ref,
                 mv_s, mi_s, acc_s):
    nb = pl.program_id(0)
    kb = pl.program_id(1)
    xt = xt_ref[...]
    w = w2_ref[...]
    s = lax.dot_general(w, xt, (((1,), (0,)), ((), ())),
                        preferred_element_type=jnp.float32)

    s1 = jnp.sum(xt * xt, axis=0, keepdims=True)
    s2 = jnp.sum(w * w, axis=1, keepdims=True)
    d = (s1 - 2.0 * s) + s2
    bmin = jnp.min(d, axis=0, keepdims=True)
    idsf = lax.broadcasted_iota(jnp.int32, (BK, BN), 0).astype(jnp.float32)
    bidx = (jnp.min(jnp.where(d == bmin, idsf, jnp.float32(3e10)),
                    axis=0, keepdims=True)
            + (kb * BK).astype(jnp.float32))

    @pl.when(kb == 0)
    def _():
        mv_s[...] = bmin
        mi_s[...] = bidx

    @pl.when(kb > 0)
    def _():
        better = bmin < mv_s[...]
        mi_s[...] = jnp.where(better, bidx, mi_s[...])
        mv_s[...] = jnp.where(better, bmin, mv_s[...])

    @pl.when(kb == KB - 1)
    def _():
        idx_ref[...] = mi_s[...].astype(jnp.int32).reshape(1, 1, BN)
        rowsum = jnp.sum(mv_s[...])

        @pl.when(nb == 0)
        def _():
            acc_s[0, 0] = rowsum

        @pl.when(nb > 0)
        def _():
            acc_s[0, 0] = acc_s[0, 0] + rowsum

        @pl.when(nb == NB - 1)
        def _():
            loss = 0.25 * acc_s[0, 0] * (1.0 / (N * D))
            loss_ref[...] = jnp.full((1, 128), loss, jnp.float32)


def _onehot_body(idx_ref, enc_ref, perp_ref, cnt_s, ent_s):
    kb = pl.program_id(0)
    nb = pl.program_id(1)
    idx_row = idx_ref[...].reshape(1, BN)
    idx_col = lax.transpose(idx_row, (1, 0))
    ids = lax.broadcasted_iota(jnp.int32, (BN, BK), 1) + kb * BK
    onehot = (ids == idx_col).astype(jnp.float32)
    enc_ref[...] = onehot
    cnt = jnp.sum(onehot, axis=0, keepdims=True)

    @pl.when(nb == 0)
    def _():
        cnt_s[...] = cnt

    @pl.when(nb > 0)
    def _():
        cnt_s[...] = cnt_s[...] + cnt

    @pl.when(nb == NB - 1)
    def _():
        p = cnt_s[...] * (1.0 / N)
        ev = jnp.sum(p * jnp.log(p + 1e-10))

        @pl.when(kb == 0)
        def _():
            ent_s[0, 0] = ev

        @pl.when(kb > 0)
        def _():
            ent_s[0, 0] = ent_s[0, 0] + ev

        @pl.when(kb == KB - 1)
        def _():
            perp_ref[...] = jnp.exp(jnp.full((1, 128), -ent_s[0, 0],
                                             jnp.float32))


def _sc_gather(table_hbm, idx_hbm, out_hbm, idx_v, rows_v, sem):
    wid = lax.axis_index("s") * _SC_NC + lax.axis_index("c")
    base = wid * _B_PER_W
    pltpu.sync_copy(idx_hbm.at[pl.ds(base, _B_PER_W)], idx_v)
    pltpu.async_copy(table_hbm.at[idx_v], rows_v, sem).wait()
    pltpu.sync_copy(rows_v, out_hbm.at[pl.ds(base, _B_PER_W)])


def kernel(inputTensor, emb_weights, ema_w, ema_cluster_size):
    del ema_w, ema_cluster_size
    flat = inputTensor.reshape(-1, D)
    xt = flat.T

    idx3, loss_out = pl.pallas_call(
        _argmin_body,
        grid=(NB, KB),
        in_specs=[
            pl.BlockSpec((D, BN), lambda nb, kb: (0, nb)),
            pl.BlockSpec((BK, D), lambda nb, kb: (kb, 0)),
        ],
        out_specs=[
            pl.BlockSpec((1, 1, BN), lambda nb, kb: (nb, 0, 0)),
            pl.BlockSpec((1, 128), lambda nb, kb: (0, 0)),
        ],
        out_shape=[
            jax.ShapeDtypeStruct((NB, 1, BN), jnp.int32),
            jax.ShapeDtypeStruct((1, 128), jnp.float32),
        ],
        scratch_shapes=[
            pltpu.VMEM((1, BN), jnp.float32),
            pltpu.VMEM((1, BN), jnp.float32),
            pltpu.SMEM((1, 1), jnp.float32),
        ],
    )(xt, emb_weights)

    idx_flat = idx3.reshape(N)

    sc_gather = functools.partial(
        pl.kernel,
        mesh=plsc.VectorSubcoreMesh(core_axis_name="c", subcore_axis_name="s"),
        out_type=jax.ShapeDtypeStruct((N, D), jnp.float32),
        scratch_types=[
            pltpu.VMEM((_B_PER_W,), jnp.int32),
            pltpu.VMEM((_B_PER_W, D), jnp.float32),
            pltpu.SemaphoreType.DMA,
        ],
    )(_sc_gather)
    quantized = sc_gather(emb_weights, idx_flat)

    enc, perp_out = pl.pallas_call(
        _onehot_body,
        grid=(KB, NB),
        in_specs=[
            pl.BlockSpec((1, 1, BN), lambda kb, nb: (nb, 0, 0)),
        ],
        out_specs=[
            pl.BlockSpec((BN, BK), lambda kb, nb: (nb, kb)),
            pl.BlockSpec((1, 128), lambda kb, nb: (0, 0)),
        ],
        out_shape=[
            jax.ShapeDtypeStruct((N, K), jnp.float32),
            jax.ShapeDtypeStruct((1, 128), jnp.float32),
        ],
        scratch_shapes=[
            pltpu.VMEM((1, BK), jnp.float32),
            pltpu.SMEM((1, 1), jnp.float32),
        ],
    )(idx3)

    loss = loss_out[0, 0]
    perplexity = perp_out[0, 0]
    quantized_st = quantized.reshape(inputTensor.shape)
    return (loss, quantized_st, perplexity, enc)
